# Initial kernel scaffold; baseline (speedup 1.0000x reference)
import jax
import jax.numpy as jnp
from jax.experimental import pallas as pl


def kernel(x, edge_index, edge_attr, bn1_gamma, bn1_beta, W1, b1, W2, b2, bn2_gamma, bn2_beta):
    raise NotImplementedError("write your pallas kernel here")


# speedup vs baseline: 4.0542x; 4.0542x over previous
import functools
import math

import jax
import jax.numpy as jnp
from jax import lax
from jax.experimental import pallas as pl
from jax.experimental.pallas import tpu as pltpu
from jax.experimental.pallas import tpu_sc as plsc

N = 10000
E = 320000
D = 128

NC = 2
NS = 16
NW = NC * NS
L = 16

EPW = E // NW
C = 80
NCHUNK = EPW // C
ROWS_PER_TILE = 640
NPAD = NS * ROWS_PER_TILE

_SC_MESH = plsc.VectorSubcoreMesh(core_axis_name="c", subcore_axis_name="s")


@functools.partial(
    pl.kernel,
    out_type=jax.ShapeDtypeStruct((NC, NPAD, D), jnp.float32),
    mesh=_SC_MESH,
    scratch_types=[
        pltpu.VMEM((C,), jnp.int32),
        pltpu.VMEM((C,), jnp.int32),
        pltpu.VMEM((C, D), jnp.float32),
        pltpu.VMEM((C, D), jnp.float32),
        pltpu.VMEM_SHARED((NPAD, D), jnp.float32),
        pltpu.SemaphoreType.DMA,
    ],
)
def _sc_aggregate(src_hbm, dst_hbm, x_hbm, ea_hbm, out_hbm,
                  sidx, didx, xrows, msg, agg, sem):
    cid = lax.axis_index("c")
    sid = lax.axis_index("s")
    wid = cid * NS + sid
    base_edge = wid * EPW
    row0 = sid * ROWS_PER_TILE

    zero = jnp.zeros((L,), jnp.float32)

    def zero_body(r, _):
        for j in range(D // L):
            msg[r, pl.ds(j * L, L)] = zero
        return 0

    lax.fori_loop(0, C, zero_body, 0)
    for k in range(ROWS_PER_TILE // C):
        pltpu.sync_copy(msg, agg.at[pl.ds(row0 + k * C, C)])
    plsc.subcore_barrier()

    def chunk_body(i, _):
        eb = pl.multiple_of(base_edge + i * C, 8)
        pltpu.sync_copy(src_hbm.at[pl.ds(eb, C)], sidx)
        pltpu.sync_copy(dst_hbm.at[pl.ds(eb, C)], didx)
        gat = pltpu.async_copy(x_hbm.at[sidx], xrows, sem)
        pltpu.sync_copy(ea_hbm.at[pl.ds(eb, C)], msg)
        gat.wait()

        def relu_body(r, _):
            for j in range(D // L):
                v = xrows[r, pl.ds(j * L, L)] + msg[r, pl.ds(j * L, L)]
                msg[r, pl.ds(j * L, L)] = jnp.maximum(v, 0.0)
            return 0

        lax.fori_loop(0, C, relu_body, 0)
        pltpu.sync_copy(msg, agg.at[didx], add=True)
        return 0

    lax.fori_loop(0, NCHUNK, chunk_body, 0)
    plsc.subcore_barrier()
    pltpu.sync_copy(agg.at[pl.ds(row0, ROWS_PER_TILE)],
                    out_hbm.at[cid, pl.ds(row0, ROWS_PER_TILE)])


def _tc_body(x_ref, aggp_ref, g1_ref, be1_ref, w1t_ref, b1_ref,
             w2t_ref, b2_ref, g2_ref, be2_ref, out_ref):
    agg = aggp_ref[0, :N, :] + aggp_ref[1, :N, :]
    h = 2.0 * x_ref[...] + agg

    m1 = jnp.mean(h, axis=0, keepdims=True)
    d1 = h - m1
    v1 = jnp.mean(d1 * d1, axis=0, keepdims=True)
    f = d1 * lax.rsqrt(v1 + 1e-5) * g1_ref[...] + be1_ref[...]

    f = jnp.dot(f, w1t_ref[...], preferred_element_type=jnp.float32)
    f = f + b1_ref[...]
    f = 0.5 * f * (1.0 + lax.erf(f * (1.0 / math.sqrt(2.0))))
    f = jnp.dot(f, w2t_ref[...], preferred_element_type=jnp.float32)
    f = f + b2_ref[...]

    z = h + f
    m2 = jnp.mean(z, axis=0, keepdims=True)
    d2 = z - m2
    v2 = jnp.mean(d2 * d2, axis=0, keepdims=True)
    out_ref[...] = d2 * lax.rsqrt(v2 + 1e-5) * g2_ref[...] + be2_ref[...]


_tc_ffn = pl.pallas_call(
    _tc_body,
    out_shape=jax.ShapeDtypeStruct((N, D), jnp.float32),
)


@jax.jit
def kernel(x, edge_index, edge_attr, bn1_gamma, bn1_beta, W1, b1, W2, b2,
           bn2_gamma, bn2_beta):
    src = edge_index[0]
    dst = edge_index[1]
    aggp = _sc_aggregate(src, dst, x, edge_attr)
    return _tc_ffn(x, aggp,
                   bn1_gamma.reshape(1, D), bn1_beta.reshape(1, D),
                   W1.T, b1.reshape(1, D),
                   W2.T, b2.reshape(1, D),
                   bn2_gamma.reshape(1, D), bn2_beta.reshape(1, D))


# speedup vs baseline: 7.0008x; 1.7268x over previous
import functools
import math

import jax
import jax.numpy as jnp
from jax import lax
from jax.experimental import pallas as pl
from jax.experimental.pallas import tpu as pltpu
from jax.experimental.pallas import tpu_sc as plsc

N = 10000
E = 320000
D = 128

NC = 2
NS = 16
NW = NC * NS
L = 16

EPW = E // NW
C = 40
NCHUNK = EPW // C
RB = 2
ROWS_PER_TILE = 640
NPAD = NS * ROWS_PER_TILE
ZCOPIES = ROWS_PER_TILE // C

_SC_MESH = plsc.VectorSubcoreMesh(core_axis_name="c", subcore_axis_name="s")


@functools.partial(
    pl.kernel,
    out_type=jax.ShapeDtypeStruct((NC, NPAD, D), jnp.float32),
    mesh=_SC_MESH,
    scratch_types=[
        pltpu.VMEM((RB, C), jnp.int32),
        pltpu.VMEM((RB, C), jnp.int32),
        pltpu.VMEM((RB, C), jnp.int32),
        pltpu.VMEM((RB, C, D), jnp.float32),
        pltpu.VMEM((RB, C, D), jnp.float32),
        pltpu.VMEM_SHARED((NPAD, D), jnp.float32),
        pltpu.SemaphoreType.DMA((RB,)),
        pltpu.SemaphoreType.DMA((RB,)),
        pltpu.SemaphoreType.DMA((RB,)),
    ],
)
def _sc_aggregate(eidx_hbm, x_hbm, ea_hbm, out_hbm,
                  pidx, sidx, didx, xr, ms, agg, sem_p, sem_l, sem_s):
    cid = lax.axis_index("c")
    sid = lax.axis_index("s")
    wid = cid * NS + sid
    base_edge = wid * EPW
    row0 = sid * ROWS_PER_TILE

    zero = jnp.zeros((L,), jnp.float32)

    def zero_body(r, _):
        for j in range(D // L):
            ms[0, r, pl.ds(j * L, L)] = zero
        return 0

    lax.fori_loop(0, C, zero_body, 0)
    zcopies = [
        pltpu.async_copy(ms.at[0], agg.at[pl.ds(row0 + k * C, C)],
                         sem_s.at[0])
        for k in range(ZCOPIES)
    ]
    for zc in zcopies:
        zc.wait()
    plsc.subcore_barrier()

    def issue_pidx(i, b):
        pltpu.async_copy(eidx_hbm.at[wid, i], pidx.at[b], sem_p.at[b])

    def wait_pidx(i, b):
        pltpu.make_async_copy(eidx_hbm.at[wid, i], pidx.at[b],
                              sem_p.at[b]).wait()

    def unpack(b):
        for o in (0, 16, 24):
            p = pidx[b, pl.ds(o, L)]
            sidx[b, pl.ds(o, L)] = p & 0xFFFF
            didx[b, pl.ds(o, L)] = jnp.right_shift(p, 16)

    def issue_load(i, b):
        eb = pl.multiple_of(base_edge + i * C, 8)
        pltpu.async_copy(x_hbm.at[sidx.at[b]], xr.at[b], sem_l.at[b])
        pltpu.async_copy(ea_hbm.at[pl.ds(eb, C)], ms.at[b], sem_l.at[b])

    def wait_load(i, b):
        eb = pl.multiple_of(base_edge + i * C, 8)
        pltpu.make_async_copy(x_hbm.at[sidx.at[b]], xr.at[b],
                              sem_l.at[b]).wait()
        pltpu.make_async_copy(ea_hbm.at[pl.ds(eb, C)], ms.at[b],
                              sem_l.at[b]).wait()

    def issue_scatter(b):
        pltpu.async_copy(ms.at[b], agg.at[didx.at[b]], sem_s.at[b],
                         add=True)

    def wait_scatter(b):
        pltpu.make_async_copy(ms.at[b], agg.at[didx.at[b]],
                              sem_s.at[b]).wait()

    def relu(b):
        def relu_body(r, _):
            for k in range(D // L):
                v = xr[b, r, pl.ds(k * L, L)] + ms[b, r, pl.ds(k * L, L)]
                ms[b, r, pl.ds(k * L, L)] = jnp.maximum(v, 0.0)
            return 0

        lax.fori_loop(0, C, relu_body, 0)

    issue_pidx(0, 0)
    wait_pidx(0, 0)
    issue_pidx(1, 1)
    unpack(0)
    issue_load(0, 0)

    def pipeline_body(t, _):
        for j in range(RB):
            i = t * RB + j
            b, bn = j, 1 - j

            @pl.when(i >= 1)
            def _():
                wait_scatter(bn)

            @pl.when(i + 1 < NCHUNK)
            def _():
                wait_pidx(i + 1, bn)
                unpack(bn)
                issue_load(i + 1, bn)

            @pl.when(i + 2 < NCHUNK)
            def _():
                issue_pidx(i + 2, b)

            wait_load(i, b)
            relu(b)
            issue_scatter(b)
        return 0

    lax.fori_loop(0, NCHUNK // RB, pipeline_body, 0)

    wait_scatter((NCHUNK - 1) % RB)
    plsc.subcore_barrier()
    pltpu.sync_copy(agg.at[pl.ds(row0, ROWS_PER_TILE)],
                    out_hbm.at[cid, pl.ds(row0, ROWS_PER_TILE)])


def _tc_body(x_ref, aggp_ref, g1_ref, be1_ref, w1t_ref, b1_ref,
             w2t_ref, b2_ref, g2_ref, be2_ref, out_ref):
    agg = aggp_ref[0, :N, :] + aggp_ref[1, :N, :]
    h = 2.0 * x_ref[...] + agg

    m1 = jnp.mean(h, axis=0, keepdims=True)
    d1 = h - m1
    v1 = jnp.mean(d1 * d1, axis=0, keepdims=True)
    f = d1 * lax.rsqrt(v1 + 1e-5) * g1_ref[...] + be1_ref[...]

    f = jnp.dot(f, w1t_ref[...], preferred_element_type=jnp.float32)
    f = f + b1_ref[...]
    f = 0.5 * f * (1.0 + lax.erf(f * (1.0 / math.sqrt(2.0))))
    f = jnp.dot(f, w2t_ref[...], preferred_element_type=jnp.float32)
    f = f + b2_ref[...]

    z = h + f
    m2 = jnp.mean(z, axis=0, keepdims=True)
    d2 = z - m2
    v2 = jnp.mean(d2 * d2, axis=0, keepdims=True)
    out_ref[...] = d2 * lax.rsqrt(v2 + 1e-5) * g2_ref[...] + be2_ref[...]


_tc_ffn = pl.pallas_call(
    _tc_body,
    out_shape=jax.ShapeDtypeStruct((N, D), jnp.float32),
)


@jax.jit
def kernel(x, edge_index, edge_attr, bn1_gamma, bn1_beta, W1, b1, W2, b2,
           bn2_gamma, bn2_beta):
    packed = edge_index[0] | (edge_index[1] << 16)
    aggp = _sc_aggregate(packed.reshape(NW, NCHUNK, C), x, edge_attr)
    return _tc_ffn(x, aggp,
                   bn1_gamma.reshape(1, D), bn1_beta.reshape(1, D),
                   W1.T, b1.reshape(1, D),
                   W2.T, b2.reshape(1, D),
                   bn2_gamma.reshape(1, D), bn2_beta.reshape(1, D))
